# Initial kernel scaffold; baseline (speedup 1.0000x reference)
import jax
import jax.numpy as jnp
from jax.experimental import pallas as pl


def kernel(x, adj, W0_1, W1_1, b1, W0_2, W1_2, b2):
    raise NotImplementedError("write your pallas kernel here")


# speedup vs baseline: 9.6961x; 9.6961x over previous
import functools

import jax
import jax.numpy as jnp
from jax import lax
from jax.experimental import pallas as pl
from jax.experimental.pallas import tpu as pltpu
from jax.experimental.pallas import tpu_sc as plsc

NC = 2
NS = 16
NW = NC * NS
LANES = 16
_K = 80
_ZC = 128


def _mesh():
    return plsc.VectorSubcoreMesh(core_axis_name="c", subcore_axis_name="s",
                                  num_cores=NC, num_subcores=NS)


def _sc_deg_body(n, e, d, src_hbm, deg_hbm, ibuf, ones_v, zbuf, acc):
    cid = lax.axis_index("c")
    sid = lax.axis_index("s")
    wid = cid * NS + sid
    ept = e // NW

    zeros = jnp.zeros((LANES,), jnp.float32)
    ones = jnp.ones((LANES,), jnp.float32)

    def f0(r, _):
        for j in range(d // LANES):
            ones_v[r, pl.ds(j * LANES, LANES)] = ones
        return 0

    lax.fori_loop(0, _K, f0, 0)

    def f1(r, _):
        for j in range(d // LANES):
            zbuf[r, pl.ds(j * LANES, LANES)] = zeros
        return 0

    lax.fori_loop(0, _ZC, f1, 0)

    nfull = n // _ZC
    nrounds = (nfull + NS - 1) // NS

    def z2(j, _):
        k = j * NS + sid

        @pl.when(k < nfull)
        def _():
            pltpu.sync_copy(zbuf, acc.at[pl.ds(k * _ZC, _ZC)])

        return 0

    lax.fori_loop(0, nrounds, z2, 0)
    tail = n - nfull * _ZC
    if tail:
        @pl.when(sid == NS - 1)
        def _():
            pltpu.sync_copy(zbuf.at[pl.ds(0, tail)],
                            acc.at[pl.ds(nfull * _ZC, tail)])
    plsc.subcore_barrier()

    def eb(b, _):
        base = wid * ept + b * _K
        pltpu.sync_copy(src_hbm.at[pl.ds(base, _K)], ibuf.at[0])
        pltpu.sync_copy(ones_v, acc.at[ibuf.at[0]], add=True)
        return 0

    lax.fori_loop(0, ept // _K, eb, 0)
    plsc.subcore_barrier()

    rpt = (n // NS) & ~7
    pltpu.sync_copy(acc.at[pl.ds(sid * rpt, rpt)],
                    deg_hbm.at[cid, pl.ds(sid * rpt, rpt)])
    wtail = n - NS * rpt
    if wtail:
        @pl.when(sid == NS - 1)
        def _():
            pltpu.sync_copy(acc.at[pl.ds(NS * rpt, wtail)],
                            deg_hbm.at[cid, pl.ds(NS * rpt, wtail)])


def _sc_deg(src, n, d=128):
    e = src.shape[0]
    return pl.kernel(
        functools.partial(_sc_deg_body, n, e, d),
        out_type=jax.ShapeDtypeStruct((NC, n, d), jnp.float32),
        mesh=_mesh(),
        scratch_types=[
            pltpu.VMEM((2, _K), jnp.int32),
            pltpu.VMEM((_K, d), jnp.float32),
            pltpu.VMEM((_ZC, d), jnp.float32),
            pltpu.VMEM_SHARED((n, d), jnp.float32),
        ],
    )(src)


def _sc_edge_body(n, e, d, src_hbm, dst_hbm, feat_hbm, out_hbm,
                  ibuf, rows, zbuf, acc):
    cid = lax.axis_index("c")
    sid = lax.axis_index("s")
    ept = e // NW

    zeros = jnp.zeros((LANES,), jnp.float32)

    def z1(r, _):
        for j in range(d // LANES):
            zbuf[r, pl.ds(j * LANES, LANES)] = zeros
        return 0

    lax.fori_loop(0, _ZC, z1, 0)

    nfull = n // _ZC
    nrounds = (nfull + NS - 1) // NS

    def z2(j, _):
        k = j * NS + sid

        @pl.when(k < nfull)
        def _():
            pltpu.sync_copy(zbuf, acc.at[pl.ds(k * _ZC, _ZC)])

        return 0

    lax.fori_loop(0, nrounds, z2, 0)
    tail = n - nfull * _ZC
    if tail:
        @pl.when(sid == NS - 1)
        def _():
            pltpu.sync_copy(zbuf.at[pl.ds(0, tail)],
                            acc.at[pl.ds(nfull * _ZC, tail)])
    plsc.subcore_barrier()

    e0 = cid * (e // NC) + sid * ept

    def eb(b, _):
        base = e0 + b * _K
        pltpu.sync_copy(src_hbm.at[pl.ds(base, _K)], ibuf.at[0])
        pltpu.sync_copy(dst_hbm.at[pl.ds(base, _K)], ibuf.at[1])
        pltpu.sync_copy(feat_hbm.at[ibuf.at[0]], rows)
        pltpu.sync_copy(rows, acc.at[ibuf.at[1]], add=True)
        return 0

    lax.fori_loop(0, ept // _K, eb, 0)
    plsc.subcore_barrier()

    rpt = (n // NS) & ~7
    pltpu.sync_copy(acc.at[pl.ds(sid * rpt, rpt)],
                    out_hbm.at[cid, pl.ds(sid * rpt, rpt)])
    wtail = n - NS * rpt
    if wtail:
        @pl.when(sid == NS - 1)
        def _():
            pltpu.sync_copy(acc.at[pl.ds(NS * rpt, wtail)],
                            out_hbm.at[cid, pl.ds(NS * rpt, wtail)])


def _sc_edge(src, dst, feat):
    n, d = feat.shape
    e = src.shape[0]
    return pl.kernel(
        functools.partial(_sc_edge_body, n, e, d),
        out_type=jax.ShapeDtypeStruct((NC, n, d), jnp.float32),
        mesh=_mesh(),
        scratch_types=[
            pltpu.VMEM((2, _K), jnp.int32),
            pltpu.VMEM((_K, d), jnp.float32),
            pltpu.VMEM((_ZC, d), jnp.float32),
            pltpu.VMEM_SHARED((n, d), jnp.float32),
        ],
    )(src, dst, feat)


def _dis_from_parts(deg_ref):
    deg = jnp.sum(deg_ref[...], axis=1)
    return jnp.where(deg > 0, lax.rsqrt(jnp.maximum(deg, 1e-12)), 0.0)


def _tc_prep_body(deg_ref, x_ref, w0_ref, xs_ref, xw0_ref):
    dis = _dis_from_parts(deg_ref)
    xb = x_ref[...]
    xs_ref[...] = xb * dis[:, None]
    xw0_ref[...] = jnp.dot(xb, w0_ref[...], preferred_element_type=jnp.float32)


def _tc_l1_body(deg_ref, xw0_ref, s_ref, w1_ref, b1_ref, w02_ref,
                hs_ref, hw2_ref):
    dis = _dis_from_parts(deg_ref)
    s = s_ref[0] + s_ref[1]
    t = jnp.dot(s, w1_ref[...], preferred_element_type=jnp.float32)
    h = jnp.maximum(xw0_ref[...] - dis[:, None] * t + b1_ref[...], 0.0)
    hs_ref[...] = h * dis[:, None]
    hw2_ref[...] = jnp.dot(h, w02_ref[...], preferred_element_type=jnp.float32)


def _tc_l2_body(deg_ref, hw2_ref, q_ref, w1_ref, b2_ref, out_ref):
    dis = _dis_from_parts(deg_ref)
    q = q_ref[0] + q_ref[1]
    t = jnp.dot(q, w1_ref[...], preferred_element_type=jnp.float32)
    out_ref[...] = jnp.maximum(hw2_ref[...] - dis[:, None] * t + b2_ref[...],
                               0.0)


_R = 2000


def _row_spec(d):
    return pl.BlockSpec((_R, d), lambda i: (i, 0))


def _full_spec(shape):
    ndim = len(shape)
    return pl.BlockSpec(shape, lambda i: (0,) * ndim)


def _tc_prep(deg_parts, x, w0):
    n, d = x.shape
    grid = (n // _R,)
    return pl.pallas_call(
        _tc_prep_body,
        grid=grid,
        in_specs=[
            pl.BlockSpec((_R, NC), lambda i: (i, 0)),
            _row_spec(d),
            _full_spec(w0.shape),
        ],
        out_specs=[_row_spec(d), _row_spec(d)],
        out_shape=[jax.ShapeDtypeStruct((n, d), jnp.float32)] * 2,
    )(deg_parts, x, w0)


def _tc_l1(deg_parts, xw0, s, w1, b1, w02):
    n, d = xw0.shape
    grid = (n // _R,)
    return pl.pallas_call(
        _tc_l1_body,
        grid=grid,
        in_specs=[
            pl.BlockSpec((_R, NC), lambda i: (i, 0)),
            _row_spec(d),
            pl.BlockSpec((NC, _R, d), lambda i: (0, i, 0)),
            _full_spec(w1.shape),
            _full_spec(b1.shape),
            _full_spec(w02.shape),
        ],
        out_specs=[_row_spec(d), _row_spec(d)],
        out_shape=[jax.ShapeDtypeStruct((n, d), jnp.float32)] * 2,
    )(deg_parts, xw0, s, w1, b1, w02)


def _tc_l2(deg_parts, hw2, q, w1, b2):
    n, d = hw2.shape
    grid = (n // _R,)
    return pl.pallas_call(
        _tc_l2_body,
        grid=grid,
        in_specs=[
            pl.BlockSpec((_R, NC), lambda i: (i, 0)),
            _row_spec(d),
            pl.BlockSpec((NC, _R, d), lambda i: (0, i, 0)),
            _full_spec(w1.shape),
            _full_spec(b2.shape),
        ],
        out_specs=_row_spec(d),
        out_shape=jax.ShapeDtypeStruct((n, d), jnp.float32),
    )(deg_parts, hw2, q, w1, b2)


def kernel(x, adj, W0_1, W1_1, b1, W0_2, W1_2, b2):
    n, d = x.shape
    adj = adj.astype(jnp.int32)
    src, dst = adj[0], adj[1]
    deg_parts = _sc_deg(src, n)[:, :, 0].T
    xs, xw0 = _tc_prep(deg_parts, x, W0_1)
    s = _sc_edge(src, dst, xs)
    hs, hw2 = _tc_l1(deg_parts, xw0, s, W1_1, b1.reshape(1, -1), W0_2)
    q = _sc_edge(src, dst, hs)
    return _tc_l2(deg_parts, hw2, q, W1_2, b2.reshape(1, -1))
